# baseline (speedup 1.0000x reference)
import jax
import jax.numpy as jnp
from jax import lax
from jax.experimental import pallas as pl
from jax.experimental.pallas import tpu as pltpu
from jax.experimental.pallas import tpu_sc as plsc

N, E, DIN, H, L, BGRAPH, M, DEDGE = 10000, 160000, 128, 16, 4, 64, 3, 4

NW = 32
CHUNK = 128
NCH = 40
EPW = CHUNK * NCH
EPAD = NW * EPW
NPAD = 10016
RPW = NPAD // 16
EB = 1024
TS = 32
TD = 16


def _h0_body(x_ref, w_ref, b_ref, p_ref, ts_ref, td_ref):
    h = x_ref[...] @ w_ref[...] + b_ref[...]
    p4 = p_ref[...]
    z12 = jnp.zeros((N, 12), jnp.float32)
    ts_ref[...] = jnp.concatenate([h, p4, z12], axis=1)
    td_ref[...] = jnp.concatenate([p4, z12], axis=1)


def _h0(x, W1, b1, pos4):
    return pl.pallas_call(
        _h0_body,
        out_shape=[
            jax.ShapeDtypeStruct((N, TS), jnp.float32),
            jax.ShapeDtypeStruct((N, TD), jnp.float32),
        ],
    )(x, W1, b1[None, :], pos4)


def _gather_body(ts_hbm, td_hbm, srcI, dstI, hp_out, pd_out,
                 idx_s, idx_d, srows0, drows0, srows1, drows1, sem0, sem1):
    c = lax.axis_index("c")
    s = lax.axis_index("s")
    wid = s * 2 + c
    pltpu.sync_copy(srcI.at[wid], idx_s)
    pltpu.sync_copy(dstI.at[wid], idx_d)

    def body(j, _):
        base = wid * EPW + j * CHUNK
        cp1 = pltpu.async_copy(ts_hbm.at[idx_s.at[j]], srows0, sem0)
        cp2 = pltpu.async_copy(td_hbm.at[idx_d.at[j]], drows0, sem1)
        cp1.wait()
        cp2.wait()
        pltpu.sync_copy(srows0, hp_out.at[pl.ds(base, CHUNK)])
        pltpu.sync_copy(drows0, pd_out.at[pl.ds(base, CHUNK)])
        return 0

    lax.fori_loop(0, NCH, body, 0)


def _gather(tableS, tableD, srcI, dstI):
    mesh = plsc.VectorSubcoreMesh(core_axis_name="c", subcore_axis_name="s")
    f = pl.kernel(
        _gather_body,
        mesh=mesh,
        compiler_params=pltpu.CompilerParams(use_tc_tiling_on_sc=False),
        out_type=[
            jax.ShapeDtypeStruct((EPAD, TS), jnp.float32),
            jax.ShapeDtypeStruct((EPAD, TD), jnp.float32),
        ],
        scratch_types=[
            pltpu.VMEM((NCH, CHUNK), jnp.int32),
            pltpu.VMEM((NCH, CHUNK), jnp.int32),
            pltpu.VMEM((CHUNK, TS), jnp.float32),
            pltpu.VMEM((CHUNK, TD), jnp.float32),
            pltpu.VMEM((CHUNK, TS), jnp.float32),
            pltpu.VMEM((CHUNK, TD), jnp.float32),
            pltpu.SemaphoreType.DMA,
            pltpu.SemaphoreType.DMA,
        ],
    )
    return f(tableS, tableD, srcI, dstI)


def _scatter_body(scat_hbm, dstI, z32, agg_out, acc, idx_d, sbuf0, sbuf1,
                  sem0, sem1):
    c = lax.axis_index("c")
    s = lax.axis_index("s")
    wid = s * 2 + c
    pltpu.sync_copy(z32.at[pl.ds(s * RPW, RPW)], acc.at[pl.ds(s * RPW, RPW)])
    pltpu.sync_copy(dstI.at[wid], idx_d)
    plsc.subcore_barrier()

    def body(j, _):
        base = wid * EPW + j * CHUNK
        pltpu.async_copy(scat_hbm.at[pl.ds(base, CHUNK)], sbuf0, sem0).wait()
        pltpu.sync_copy(sbuf0, acc.at[idx_d.at[j]], add=True)
        return 0

    lax.fori_loop(0, NCH, body, 0)
    plsc.subcore_barrier()
    pltpu.sync_copy(acc.at[pl.ds(s * RPW, RPW)],
                    agg_out.at[c, pl.ds(s * RPW, RPW)])


def _scatter(scat, dstI, z32):
    mesh = plsc.VectorSubcoreMesh(core_axis_name="c", subcore_axis_name="s")
    f = pl.kernel(
        _scatter_body,
        mesh=mesh,
        compiler_params=pltpu.CompilerParams(use_tc_tiling_on_sc=False),
        out_type=[
            jax.ShapeDtypeStruct((2, NPAD, TS), jnp.float32),
        ],
        scratch_types=[
            pltpu.VMEM_SHARED((NPAD, TS), jnp.float32),
            pltpu.VMEM((NCH, CHUNK), jnp.int32),
            pltpu.VMEM((CHUNK, TS), jnp.float32),
            pltpu.VMEM((CHUNK, TS), jnp.float32),
            pltpu.SemaphoreType.DMA,
            pltpu.SemaphoreType.DMA,
        ],
    )
    return f(scat, dstI, z32)[0]


def _edge_body(ea_ref, hp_ref, pd_ref, nw1_ref, nb1_ref, bmat_ref,
               rmat_ref, nb2t_ref, wc_ref, scat_ref):
    gid = pl.program_id(0)
    eidx = gid * EB + lax.broadcasted_iota(jnp.int32, (EB, 1), 0)
    valid = (eidx < E).astype(jnp.float32)

    hp = hp_ref[...]
    hs = hp[:, 0:16]
    rel = hp[:, 16:20] - pd_ref[:, 0:4]
    dist = jnp.sqrt(jnp.sum(rel * rel, axis=1, keepdims=True) + 1e-12)
    pre = ea_ref[...] @ nw1_ref[0:4, :] + dist * nw1_ref[4:5, :] + nb1_ref[...]
    t = pre * (1.0 / (1.0 + jnp.exp(-pre)))
    trep = jnp.broadcast_to(t[:, :, None], (EB, 32, H)).reshape(EB, 32 * H)
    me = trep * (hs @ bmat_ref[...])
    m = me[:, :256] + me[:, 256:]
    m = m[:, :128] + m[:, 128:]
    m = m[:, :64] + m[:, 64:]
    m = m[:, :32] + m[:, 32:]
    m = m[:, :16] + m[:, 16:]
    msg = (m + hs @ nb2t_ref[...]) * valid
    cw = jnp.sum(msg * wc_ref[...], axis=1, keepdims=True)
    col = lax.broadcasted_iota(jnp.int32, (1, 4), 1)
    rc = rel * cw + jnp.where(col == 3, 1.0, 0.0) * valid
    z12 = jnp.zeros((EB, 12), jnp.float32)
    scat_ref[...] = jnp.concatenate([msg, rc, z12], axis=1)


def _edge_net(ea, hp_src, p_dst, nW1, nb1, Bmat, Rmat, nb2T, wc):
    grid = EPAD // EB
    return pl.pallas_call(
        _edge_body,
        grid=(grid,),
        in_specs=[
            pl.BlockSpec((EB, 4), lambda i: (i, 0)),
            pl.BlockSpec((EB, TS), lambda i: (i, 0)),
            pl.BlockSpec((EB, TD), lambda i: (i, 0)),
            pl.BlockSpec((5, 32), lambda i: (0, 0)),
            pl.BlockSpec((1, 32), lambda i: (0, 0)),
            pl.BlockSpec((H, 512), lambda i: (0, 0)),
            pl.BlockSpec((32, 512), lambda i: (0, 0)),
            pl.BlockSpec((H, H), lambda i: (0, 0)),
            pl.BlockSpec((1, H), lambda i: (0, 0)),
        ],
        out_specs=[
            pl.BlockSpec((EB, TS), lambda i: (i, 0)),
        ],
        out_shape=[
            jax.ShapeDtypeStruct((EPAD, TS), jnp.float32),
        ],
    )(ea, hp_src, p_dst, nW1, nb1, Bmat, Rmat, nb2T, wc)[0]


def _update_body(ts_ref, agg_ref, wr_ref, br_ref, tso_ref, tdo_ref):
    acc = agg_ref[0, :N, :] + agg_ref[1, :N, :]
    agg = acc[:, 0:16]
    ps = acc[:, 16:20]
    deg = jnp.maximum(ps[:, 3:4], 1.0)
    h = ts_ref[:, 0:16]
    pos4 = ts_ref[:, 16:20]
    h_new = h + h @ wr_ref[...] + agg / deg + br_ref[...]
    col = lax.broadcasted_iota(jnp.int32, (1, 4), 1)
    mask = jnp.where(col < 3, 1.0, 0.0)
    p_new = pos4 + (ps * mask) / deg
    z12 = jnp.zeros((N, 12), jnp.float32)
    tso_ref[...] = jnp.concatenate([h_new, p_new, z12], axis=1)
    tdo_ref[...] = jnp.concatenate([p_new, z12], axis=1)


def _update(tableS, agg2, Wr_l, br_l):
    return pl.pallas_call(
        _update_body,
        out_shape=[
            jax.ShapeDtypeStruct((N, TS), jnp.float32),
            jax.ShapeDtypeStruct((N, TD), jnp.float32),
        ],
    )(tableS, agg2, Wr_l, br_l[None, :])


def _s2s_body(ts_ref, b_ref, wih_ref, whh_ref, bl_ref, wo1_ref, bo1_ref,
              wo2_ref, bo2_ref, out_ref):
    h = ts_ref[:, 0:16]
    bidx = b_ref[...]
    gcol = lax.broadcasted_iota(jnp.int32, (N, BGRAPH), 1)
    onehot = (bidx == gcol).astype(jnp.float32)

    def sig(v):
        return 1.0 / (1.0 + jnp.exp(-v))

    q_star = jnp.zeros((BGRAPH, 2 * H), jnp.float32)
    hs = jnp.zeros((BGRAPH, H), jnp.float32)
    cs = jnp.zeros((BGRAPH, H), jnp.float32)
    dn0 = (((0,), (0,)), ((), ()))
    for _ in range(M):
        gates = q_star @ wih_ref[...] + hs @ whh_ref[...] + bl_ref[...]
        i = gates[:, 0 * H:1 * H]
        f = gates[:, 1 * H:2 * H]
        g = gates[:, 2 * H:3 * H]
        o = gates[:, 3 * H:4 * H]
        cs = sig(f) * cs + sig(i) * jnp.tanh(g)
        hs = sig(o) * jnp.tanh(cs)
        qb = onehot @ hs
        escore = jnp.sum(h * qb, axis=1, keepdims=True)
        masked = jnp.where(onehot > 0.0, escore, -3.4e38)
        emax = jnp.max(masked, axis=0, keepdims=True)
        emax = jnp.where(emax < -1e37, 0.0, emax)
        a = jnp.exp(escore - onehot @ emax.T)
        asum = lax.dot_general(onehot, a, dn0)
        asum = jnp.where(asum > 0.0, asum, 1.0)
        anorm = a / (onehot @ asum)
        r = lax.dot_general(onehot, anorm * h, dn0)
        q_star = jnp.concatenate([hs, r], axis=1)
    u = q_star @ wo1_ref[...] + bo1_ref[...]
    u = u * sig(u)
    out_ref[...] = u @ wo2_ref[...] + bo2_ref[...]


def _set2set(tableS, batch, W_ih, W_hh, b_lstm, Wo1, bo1, Wo2, bo2):
    return pl.pallas_call(
        _s2s_body,
        out_shape=jax.ShapeDtypeStruct((BGRAPH, 1), jnp.float32),
    )(tableS, batch[:, None], W_ih, W_hh, b_lstm[None, :], Wo1, bo1[None, :],
      Wo2, bo2[None, :])


def kernel(x, edge_index, edge_attr, pos, batch, W1, b1, nW1, nb1, nW2, nb2,
           Wr, br, Wc, W_ih, W_hh, b_lstm, Wo1, bo1, Wo2, bo2):
    src = edge_index[0].astype(jnp.int32)
    dst = edge_index[1].astype(jnp.int32)
    srcI = jnp.pad(src, (0, EPAD - E)).reshape(NW, NCH, CHUNK)
    dstI = jnp.pad(dst, (0, EPAD - E)).reshape(NW, NCH, CHUNK)
    ea = jnp.pad(edge_attr, ((0, EPAD - E), (0, 0)))
    pos4 = jnp.pad(pos, ((0, 0), (0, 1)))
    z32 = jnp.zeros((NPAD, TS), jnp.float32)
    Bmat = jnp.transpose(nW2.reshape(32, H, H), (2, 0, 1)).reshape(H, 32 * H)
    Rmat = jnp.repeat(jnp.eye(32, dtype=jnp.float32), H, axis=1)
    nb2T = nb2.reshape(H, H).T

    tableS, tableD = _h0(x, W1, b1, pos4)
    for l in range(L):
        hp_src, p_dst = _gather(tableS, tableD, srcI, dstI)
        scat = _edge_net(ea, hp_src, p_dst, nW1, nb1[None, :],
                         Bmat, Rmat, nb2T, Wc[l].reshape(1, H))
        agg2 = _scatter(scat, dstI, z32)
        tableS, tableD = _update(tableS, agg2, Wr[l], br[l])
    out = _set2set(tableS, batch.astype(jnp.int32), W_ih, W_hh, b_lstm,
                   Wo1, bo1, Wo2, bo2)
    return out.reshape(-1)


# speedup vs baseline: 2.6016x; 2.6016x over previous
import jax
import jax.numpy as jnp
from jax import lax
from jax.experimental import pallas as pl
from jax.experimental.pallas import tpu as pltpu
from jax.experimental.pallas import tpu_sc as plsc

N, E, DIN, H, L, BGRAPH, M, DEDGE = 10000, 160000, 128, 16, 4, 64, 3, 4

NW = 32
CHUNK = 128
NCH = 40
EPW = CHUNK * NCH
EPAD = NW * EPW
NPAD = 10016
RPW = NPAD // 16
EB = 1024
TS = 32
TD = 16


def _h0_body(x_ref, w_ref, b_ref, p_ref, ts_ref, td_ref):
    h = x_ref[...] @ w_ref[...] + b_ref[...]
    p4 = p_ref[...]
    z12 = jnp.zeros((N, 12), jnp.float32)
    ts_ref[...] = jnp.concatenate([h, p4, z12], axis=1)
    td_ref[...] = jnp.concatenate([p4, z12], axis=1)


def _h0(x, W1, b1, pos4):
    return pl.pallas_call(
        _h0_body,
        out_shape=[
            jax.ShapeDtypeStruct((N, TS), jnp.float32),
            jax.ShapeDtypeStruct((N, TD), jnp.float32),
        ],
    )(x, W1, b1[None, :], pos4)


def _gather_body(ts_hbm, td_hbm, srcI, dstI, hp_out, pd_out,
                 idx_s, idx_d, srows0, drows0, srows1, drows1, sem0, sem1):
    c = lax.axis_index("c")
    s = lax.axis_index("s")
    wid = s * 2 + c
    pltpu.sync_copy(srcI.at[wid], idx_s)
    pltpu.sync_copy(dstI.at[wid], idx_d)

    def body(j, _):
        base = wid * EPW + j * CHUNK
        cp1 = pltpu.async_copy(ts_hbm.at[idx_s.at[j]], srows0, sem0)
        cp2 = pltpu.async_copy(td_hbm.at[idx_d.at[j]], drows0, sem1)
        cp1.wait()
        cp2.wait()
        pltpu.sync_copy(srows0, hp_out.at[pl.ds(base, CHUNK)])
        pltpu.sync_copy(drows0, pd_out.at[pl.ds(base, CHUNK)])
        return 0

    lax.fori_loop(0, NCH, body, 0)


def _gather(tableS, tableD, srcI, dstI):
    mesh = plsc.VectorSubcoreMesh(core_axis_name="c", subcore_axis_name="s")
    f = pl.kernel(
        _gather_body,
        mesh=mesh,
        compiler_params=pltpu.CompilerParams(use_tc_tiling_on_sc=False),
        out_type=[
            jax.ShapeDtypeStruct((EPAD, TS), jnp.float32),
            jax.ShapeDtypeStruct((EPAD, TD), jnp.float32),
        ],
        scratch_types=[
            pltpu.VMEM((NCH, CHUNK), jnp.int32),
            pltpu.VMEM((NCH, CHUNK), jnp.int32),
            pltpu.VMEM((CHUNK, TS), jnp.float32),
            pltpu.VMEM((CHUNK, TD), jnp.float32),
            pltpu.VMEM((CHUNK, TS), jnp.float32),
            pltpu.VMEM((CHUNK, TD), jnp.float32),
            pltpu.SemaphoreType.DMA,
            pltpu.SemaphoreType.DMA,
        ],
    )
    return f(tableS, tableD, srcI, dstI)


def _scatter_body(scat_hbm, dstI, z32, agg_out, acc, idx_d, sbuf0, sbuf1,
                  sem0, sem1):
    c = lax.axis_index("c")
    s = lax.axis_index("s")
    wid = s * 2 + c
    pltpu.sync_copy(z32.at[pl.ds(s * RPW, RPW)], acc.at[pl.ds(s * RPW, RPW)])
    pltpu.sync_copy(dstI.at[wid], idx_d)
    plsc.subcore_barrier()

    def body(j, _):
        base = wid * EPW + j * CHUNK
        pltpu.async_copy(scat_hbm.at[pl.ds(base, CHUNK)], sbuf0, sem0).wait()
        pltpu.sync_copy(sbuf0, acc.at[idx_d.at[j]], add=True)
        return 0

    lax.fori_loop(0, NCH, body, 0)
    plsc.subcore_barrier()
    pltpu.sync_copy(acc.at[pl.ds(s * RPW, RPW)],
                    agg_out.at[c, pl.ds(s * RPW, RPW)])


def _scatter(scat, dstI, z32):
    mesh = plsc.VectorSubcoreMesh(core_axis_name="c", subcore_axis_name="s")
    f = pl.kernel(
        _scatter_body,
        mesh=mesh,
        compiler_params=pltpu.CompilerParams(use_tc_tiling_on_sc=False),
        out_type=[
            jax.ShapeDtypeStruct((2, NPAD, TS), jnp.float32),
        ],
        scratch_types=[
            pltpu.VMEM_SHARED((NPAD, TS), jnp.float32),
            pltpu.VMEM((NCH, CHUNK), jnp.int32),
            pltpu.VMEM((CHUNK, TS), jnp.float32),
            pltpu.VMEM((CHUNK, TS), jnp.float32),
            pltpu.SemaphoreType.DMA,
            pltpu.SemaphoreType.DMA,
        ],
    )
    return f(scat, dstI, z32)[0]


def _edge_body(ea_ref, hp_ref, pd_ref, nw1_ref, nb1_ref, bmat_ref,
               rmat_ref, nb2t_ref, wc_ref, scat_ref):
    gid = pl.program_id(0)
    eidx = gid * EB + lax.broadcasted_iota(jnp.int32, (EB, 1), 0)
    valid = (eidx < E).astype(jnp.float32)

    hp = hp_ref[...]
    hs = hp[:, 0:16]
    rel = hp[:, 16:20] - pd_ref[:, 0:4]
    dist = jnp.sqrt(jnp.sum(rel * rel, axis=1, keepdims=True) + 1e-12)
    pre = ea_ref[...] @ nw1_ref[0:4, :] + dist * nw1_ref[4:5, :] + nb1_ref[...]
    t = pre * (1.0 / (1.0 + jnp.exp(-pre)))
    me = (t @ rmat_ref[...]) * (hs @ bmat_ref[...])
    m = me[:, :256] + me[:, 256:]
    m = m[:, :128] + m[:, 128:]
    m = m[:, :64] + m[:, 64:]
    m = m[:, :32] + m[:, 32:]
    m = m[:, :16] + m[:, 16:]
    msg = (m + hs @ nb2t_ref[...]) * valid
    cw = jnp.sum(msg * wc_ref[...], axis=1, keepdims=True)
    col = lax.broadcasted_iota(jnp.int32, (1, 4), 1)
    rc = rel * cw + jnp.where(col == 3, 1.0, 0.0) * valid
    z12 = jnp.zeros((EB, 12), jnp.float32)
    scat_ref[...] = jnp.concatenate([msg, rc, z12], axis=1)


def _edge_net(ea, hp_src, p_dst, nW1, nb1, Bmat, Rmat, nb2T, wc):
    grid = EPAD // EB
    return pl.pallas_call(
        _edge_body,
        grid=(grid,),
        in_specs=[
            pl.BlockSpec((EB, 4), lambda i: (i, 0)),
            pl.BlockSpec((EB, TS), lambda i: (i, 0)),
            pl.BlockSpec((EB, TD), lambda i: (i, 0)),
            pl.BlockSpec((5, 32), lambda i: (0, 0)),
            pl.BlockSpec((1, 32), lambda i: (0, 0)),
            pl.BlockSpec((H, 512), lambda i: (0, 0)),
            pl.BlockSpec((32, 512), lambda i: (0, 0)),
            pl.BlockSpec((H, H), lambda i: (0, 0)),
            pl.BlockSpec((1, H), lambda i: (0, 0)),
        ],
        out_specs=[
            pl.BlockSpec((EB, TS), lambda i: (i, 0)),
        ],
        out_shape=[
            jax.ShapeDtypeStruct((EPAD, TS), jnp.float32),
        ],
    )(ea, hp_src, p_dst, nW1, nb1, Bmat, Rmat, nb2T, wc)[0]


def _update_body(ts_ref, agg_ref, wr_ref, br_ref, tso_ref, tdo_ref):
    acc = agg_ref[0, :N, :] + agg_ref[1, :N, :]
    agg = acc[:, 0:16]
    ps = acc[:, 16:20]
    deg = jnp.maximum(ps[:, 3:4], 1.0)
    h = ts_ref[:, 0:16]
    pos4 = ts_ref[:, 16:20]
    h_new = h + h @ wr_ref[...] + agg / deg + br_ref[...]
    col = lax.broadcasted_iota(jnp.int32, (1, 4), 1)
    mask = jnp.where(col < 3, 1.0, 0.0)
    p_new = pos4 + (ps * mask) / deg
    z12 = jnp.zeros((N, 12), jnp.float32)
    tso_ref[...] = jnp.concatenate([h_new, p_new, z12], axis=1)
    tdo_ref[...] = jnp.concatenate([p_new, z12], axis=1)


def _update(tableS, agg2, Wr_l, br_l):
    return pl.pallas_call(
        _update_body,
        out_shape=[
            jax.ShapeDtypeStruct((N, TS), jnp.float32),
            jax.ShapeDtypeStruct((N, TD), jnp.float32),
        ],
    )(tableS, agg2, Wr_l, br_l[None, :])


def _s2s_body(ts_ref, b_ref, wih_ref, whh_ref, bl_ref, wo1_ref, bo1_ref,
              wo2_ref, bo2_ref, out_ref):
    h = ts_ref[:, 0:16]
    bidx = b_ref[...]
    gcol = lax.broadcasted_iota(jnp.int32, (N, BGRAPH), 1)
    onehot = (bidx == gcol).astype(jnp.float32)

    def sig(v):
        return 1.0 / (1.0 + jnp.exp(-v))

    q_star = jnp.zeros((BGRAPH, 2 * H), jnp.float32)
    hs = jnp.zeros((BGRAPH, H), jnp.float32)
    cs = jnp.zeros((BGRAPH, H), jnp.float32)
    dn0 = (((0,), (0,)), ((), ()))
    for _ in range(M):
        gates = q_star @ wih_ref[...] + hs @ whh_ref[...] + bl_ref[...]
        i = gates[:, 0 * H:1 * H]
        f = gates[:, 1 * H:2 * H]
        g = gates[:, 2 * H:3 * H]
        o = gates[:, 3 * H:4 * H]
        cs = sig(f) * cs + sig(i) * jnp.tanh(g)
        hs = sig(o) * jnp.tanh(cs)
        qb = onehot @ hs
        escore = jnp.sum(h * qb, axis=1, keepdims=True)
        masked = jnp.where(onehot > 0.0, escore, -3.4e38)
        emax = jnp.max(masked, axis=0, keepdims=True)
        emax = jnp.where(emax < -1e37, 0.0, emax)
        a = jnp.exp(escore - onehot @ emax.T)
        asum = lax.dot_general(onehot, a, dn0)
        asum = jnp.where(asum > 0.0, asum, 1.0)
        anorm = a / (onehot @ asum)
        r = lax.dot_general(onehot, anorm * h, dn0)
        q_star = jnp.concatenate([hs, r], axis=1)
    u = q_star @ wo1_ref[...] + bo1_ref[...]
    u = u * sig(u)
    out_ref[...] = u @ wo2_ref[...] + bo2_ref[...]


def _set2set(tableS, batch, W_ih, W_hh, b_lstm, Wo1, bo1, Wo2, bo2):
    return pl.pallas_call(
        _s2s_body,
        out_shape=jax.ShapeDtypeStruct((BGRAPH, 1), jnp.float32),
    )(tableS, batch[:, None], W_ih, W_hh, b_lstm[None, :], Wo1, bo1[None, :],
      Wo2, bo2[None, :])


def kernel(x, edge_index, edge_attr, pos, batch, W1, b1, nW1, nb1, nW2, nb2,
           Wr, br, Wc, W_ih, W_hh, b_lstm, Wo1, bo1, Wo2, bo2):
    src = edge_index[0].astype(jnp.int32)
    dst = edge_index[1].astype(jnp.int32)
    srcI = jnp.pad(src, (0, EPAD - E)).reshape(NW, NCH, CHUNK)
    dstI = jnp.pad(dst, (0, EPAD - E)).reshape(NW, NCH, CHUNK)
    ea = jnp.pad(edge_attr, ((0, EPAD - E), (0, 0)))
    pos4 = jnp.pad(pos, ((0, 0), (0, 1)))
    z32 = jnp.zeros((NPAD, TS), jnp.float32)
    Bmat = jnp.transpose(nW2.reshape(32, H, H), (2, 0, 1)).reshape(H, 32 * H)
    Rmat = jnp.repeat(jnp.eye(32, dtype=jnp.float32), H, axis=1)
    nb2T = nb2.reshape(H, H).T

    tableS, tableD = _h0(x, W1, b1, pos4)
    for l in range(L):
        hp_src, p_dst = _gather(tableS, tableD, srcI, dstI)
        scat = _edge_net(ea, hp_src, p_dst, nW1, nb1[None, :],
                         Bmat, Rmat, nb2T, Wc[l].reshape(1, H))
        agg2 = _scatter(scat, dstI, z32)
        tableS, tableD = _update(tableS, agg2, Wr[l], br[l])
    out = _set2set(tableS, batch.astype(jnp.int32), W_ih, W_hh, b_lstm,
                   Wo1, bo1, Wo2, bo2)
    return out.reshape(-1)


# speedup vs baseline: 2.7231x; 1.0467x over previous
import jax
import jax.numpy as jnp
from jax import lax
from jax.experimental import pallas as pl
from jax.experimental.pallas import tpu as pltpu
from jax.experimental.pallas import tpu_sc as plsc

N, E, DIN, H, L, BGRAPH, M, DEDGE = 10000, 160000, 128, 16, 4, 64, 3, 4

NW = 32
CHUNK = 128
NCH = 40
EPW = CHUNK * NCH
NG = 4
GE = EPW // NG
EPAD = NW * EPW
NPAD = 10016
RPW = NPAD // 16
EB = 1024
TS = 32
TD = 16


def _h0_body(x_ref, w_ref, b_ref, p_ref, ts_ref, td_ref):
    h = x_ref[...] @ w_ref[...] + b_ref[...]
    p4 = p_ref[...]
    z12 = jnp.zeros((N, 12), jnp.float32)
    ts_ref[...] = jnp.concatenate([h, p4, z12], axis=1)
    td_ref[...] = jnp.concatenate([p4, z12], axis=1)


def _h0(x, W1, b1, pos4):
    return pl.pallas_call(
        _h0_body,
        out_shape=[
            jax.ShapeDtypeStruct((N, TS), jnp.float32),
            jax.ShapeDtypeStruct((N, TD), jnp.float32),
        ],
    )(x, W1, b1[None, :], pos4)


def _gather_body(ts_hbm, td_hbm, srcI, dstI, hp_out, pd_out,
                 idx_s, idx_d, srows0, drows0, sem0, sem1):
    c = lax.axis_index("c")
    s = lax.axis_index("s")
    wid = s * 2 + c
    pltpu.sync_copy(srcI.at[wid], idx_s)
    pltpu.sync_copy(dstI.at[wid], idx_d)

    def issue(g, sbuf, dbuf, sem):
        pltpu.async_copy(ts_hbm.at[idx_s.at[g]], sbuf, sem)
        pltpu.async_copy(td_hbm.at[idx_d.at[g]], dbuf, sem)

    def drain_write(g, sbuf, dbuf, sem):
        base = wid * EPW + g * GE
        pltpu.make_async_copy(ts_hbm.at[idx_s.at[g]], sbuf, sem).wait()
        pltpu.make_async_copy(td_hbm.at[idx_d.at[g]], dbuf, sem).wait()
        pltpu.sync_copy(sbuf, hp_out.at[pl.ds(base, GE)])
        pltpu.sync_copy(dbuf, pd_out.at[pl.ds(base, GE)])

    def body(g, _):
        issue(g, srows0, drows0, sem0)
        drain_write(g, srows0, drows0, sem0)
        return 0

    lax.fori_loop(0, NG, body, 0)


def _gather(tableS, tableD, srcI, dstI):
    mesh = plsc.VectorSubcoreMesh(core_axis_name="c", subcore_axis_name="s")
    f = pl.kernel(
        _gather_body,
        mesh=mesh,
        compiler_params=pltpu.CompilerParams(use_tc_tiling_on_sc=False),
        out_type=[
            jax.ShapeDtypeStruct((EPAD, TS), jnp.float32),
            jax.ShapeDtypeStruct((EPAD, TD), jnp.float32),
        ],
        scratch_types=[
            pltpu.VMEM((NG, GE), jnp.int32),
            pltpu.VMEM((NG, GE), jnp.int32),
            pltpu.VMEM((GE, TS), jnp.float32),
            pltpu.VMEM((GE, TD), jnp.float32),
            pltpu.SemaphoreType.DMA,
            pltpu.SemaphoreType.DMA,
        ],
    )
    return f(tableS, tableD, srcI, dstI)


def _scatter_body(scat_hbm, dstI, z32, agg_out, acc, idx_d, sbuf0, sem0):
    c = lax.axis_index("c")
    s = lax.axis_index("s")
    wid = s * 2 + c
    pltpu.sync_copy(z32.at[pl.ds(s * RPW, RPW)], acc.at[pl.ds(s * RPW, RPW)])
    pltpu.sync_copy(dstI.at[wid], idx_d)
    plsc.subcore_barrier()

    def issue(g, buf, sem):
        base = wid * EPW + g * GE
        pltpu.async_copy(scat_hbm.at[pl.ds(base, GE)], buf, sem)

    def drain_add(g, buf, sem):
        base = wid * EPW + g * GE
        pltpu.make_async_copy(scat_hbm.at[pl.ds(base, GE)], buf, sem).wait()
        pltpu.sync_copy(buf, acc.at[idx_d.at[g]], add=True)

    def body(g, _):
        issue(g, sbuf0, sem0)
        drain_add(g, sbuf0, sem0)
        return 0

    lax.fori_loop(0, NG, body, 0)
    plsc.subcore_barrier()
    pltpu.sync_copy(acc.at[pl.ds(s * RPW, RPW)],
                    agg_out.at[c, pl.ds(s * RPW, RPW)])


def _scatter(scat, dstI, z32):
    mesh = plsc.VectorSubcoreMesh(core_axis_name="c", subcore_axis_name="s")
    f = pl.kernel(
        _scatter_body,
        mesh=mesh,
        compiler_params=pltpu.CompilerParams(use_tc_tiling_on_sc=False),
        out_type=[
            jax.ShapeDtypeStruct((2, NPAD, TS), jnp.float32),
        ],
        scratch_types=[
            pltpu.VMEM_SHARED((NPAD, TS), jnp.float32),
            pltpu.VMEM((NG, GE), jnp.int32),
            pltpu.VMEM((GE, TS), jnp.float32),
            pltpu.SemaphoreType.DMA,
        ],
    )
    return f(scat, dstI, z32)[0]


def _edge_body(ea_ref, hp_ref, pd_ref, nw1_ref, nb1_ref, bmat_ref,
               rmat_ref, nb2t_ref, wc_ref, scat_ref):
    gid = pl.program_id(0)
    eidx = gid * EB + lax.broadcasted_iota(jnp.int32, (EB, 1), 0)
    valid = (eidx < E).astype(jnp.float32)

    hp = hp_ref[...]
    hs = hp[:, 0:16]
    rel = hp[:, 16:20] - pd_ref[:, 0:4]
    dist = jnp.sqrt(jnp.sum(rel * rel, axis=1, keepdims=True) + 1e-12)
    pre = ea_ref[...] @ nw1_ref[0:4, :] + dist * nw1_ref[4:5, :] + nb1_ref[...]
    t = pre * (1.0 / (1.0 + jnp.exp(-pre)))
    me = (t @ rmat_ref[...]) * (hs @ bmat_ref[...])
    m = me[:, :256] + me[:, 256:]
    m = m[:, :128] + m[:, 128:]
    m = m[:, :64] + m[:, 64:]
    m = m[:, :32] + m[:, 32:]
    m = m[:, :16] + m[:, 16:]
    msg = (m + hs @ nb2t_ref[...]) * valid
    cw = jnp.sum(msg * wc_ref[...], axis=1, keepdims=True)
    col = lax.broadcasted_iota(jnp.int32, (1, 4), 1)
    rc = rel * cw + jnp.where(col == 3, 1.0, 0.0) * valid
    z12 = jnp.zeros((EB, 12), jnp.float32)
    scat_ref[...] = jnp.concatenate([msg, rc, z12], axis=1)


def _edge_net(ea, hp_src, p_dst, nW1, nb1, Bmat, Rmat, nb2T, wc):
    grid = EPAD // EB
    return pl.pallas_call(
        _edge_body,
        grid=(grid,),
        in_specs=[
            pl.BlockSpec((EB, 4), lambda i: (i, 0)),
            pl.BlockSpec((EB, TS), lambda i: (i, 0)),
            pl.BlockSpec((EB, TD), lambda i: (i, 0)),
            pl.BlockSpec((5, 32), lambda i: (0, 0)),
            pl.BlockSpec((1, 32), lambda i: (0, 0)),
            pl.BlockSpec((H, 512), lambda i: (0, 0)),
            pl.BlockSpec((32, 512), lambda i: (0, 0)),
            pl.BlockSpec((H, H), lambda i: (0, 0)),
            pl.BlockSpec((1, H), lambda i: (0, 0)),
        ],
        out_specs=[
            pl.BlockSpec((EB, TS), lambda i: (i, 0)),
        ],
        out_shape=[
            jax.ShapeDtypeStruct((EPAD, TS), jnp.float32),
        ],
    )(ea, hp_src, p_dst, nW1, nb1, Bmat, Rmat, nb2T, wc)[0]


def _update_body(ts_ref, agg_ref, wr_ref, br_ref, tso_ref, tdo_ref):
    acc = agg_ref[0, :N, :] + agg_ref[1, :N, :]
    agg = acc[:, 0:16]
    ps = acc[:, 16:20]
    deg = jnp.maximum(ps[:, 3:4], 1.0)
    h = ts_ref[:, 0:16]
    pos4 = ts_ref[:, 16:20]
    h_new = h + h @ wr_ref[...] + agg / deg + br_ref[...]
    col = lax.broadcasted_iota(jnp.int32, (1, 4), 1)
    mask = jnp.where(col < 3, 1.0, 0.0)
    p_new = pos4 + (ps * mask) / deg
    z12 = jnp.zeros((N, 12), jnp.float32)
    tso_ref[...] = jnp.concatenate([h_new, p_new, z12], axis=1)
    tdo_ref[...] = jnp.concatenate([p_new, z12], axis=1)


def _update(tableS, agg2, Wr_l, br_l):
    return pl.pallas_call(
        _update_body,
        out_shape=[
            jax.ShapeDtypeStruct((N, TS), jnp.float32),
            jax.ShapeDtypeStruct((N, TD), jnp.float32),
        ],
    )(tableS, agg2, Wr_l, br_l[None, :])


def _s2s_body(ts_ref, b_ref, wih_ref, whh_ref, bl_ref, wo1_ref, bo1_ref,
              wo2_ref, bo2_ref, out_ref):
    h = ts_ref[:, 0:16]
    bidx = b_ref[...]
    gcol = lax.broadcasted_iota(jnp.int32, (N, BGRAPH), 1)
    onehot = (bidx == gcol).astype(jnp.float32)

    def sig(v):
        return 1.0 / (1.0 + jnp.exp(-v))

    q_star = jnp.zeros((BGRAPH, 2 * H), jnp.float32)
    hs = jnp.zeros((BGRAPH, H), jnp.float32)
    cs = jnp.zeros((BGRAPH, H), jnp.float32)
    dn0 = (((0,), (0,)), ((), ()))
    for _ in range(M):
        gates = q_star @ wih_ref[...] + hs @ whh_ref[...] + bl_ref[...]
        i = gates[:, 0 * H:1 * H]
        f = gates[:, 1 * H:2 * H]
        g = gates[:, 2 * H:3 * H]
        o = gates[:, 3 * H:4 * H]
        cs = sig(f) * cs + sig(i) * jnp.tanh(g)
        hs = sig(o) * jnp.tanh(cs)
        qb = onehot @ hs
        escore = jnp.sum(h * qb, axis=1, keepdims=True)
        masked = jnp.where(onehot > 0.0, escore, -3.4e38)
        emax = jnp.max(masked, axis=0, keepdims=True)
        emax = jnp.where(emax < -1e37, 0.0, emax)
        a = jnp.exp(escore - onehot @ emax.T)
        asum = lax.dot_general(onehot, a, dn0)
        asum = jnp.where(asum > 0.0, asum, 1.0)
        anorm = a / (onehot @ asum)
        r = lax.dot_general(onehot, anorm * h, dn0)
        q_star = jnp.concatenate([hs, r], axis=1)
    u = q_star @ wo1_ref[...] + bo1_ref[...]
    u = u * sig(u)
    out_ref[...] = u @ wo2_ref[...] + bo2_ref[...]


def _set2set(tableS, batch, W_ih, W_hh, b_lstm, Wo1, bo1, Wo2, bo2):
    return pl.pallas_call(
        _s2s_body,
        out_shape=jax.ShapeDtypeStruct((BGRAPH, 1), jnp.float32),
    )(tableS, batch[:, None], W_ih, W_hh, b_lstm[None, :], Wo1, bo1[None, :],
      Wo2, bo2[None, :])


def kernel(x, edge_index, edge_attr, pos, batch, W1, b1, nW1, nb1, nW2, nb2,
           Wr, br, Wc, W_ih, W_hh, b_lstm, Wo1, bo1, Wo2, bo2):
    src = edge_index[0].astype(jnp.int32)
    dst = edge_index[1].astype(jnp.int32)
    srcI = jnp.pad(src, (0, EPAD - E)).reshape(NW, NG, GE)
    dstI = jnp.pad(dst, (0, EPAD - E)).reshape(NW, NG, GE)
    ea = jnp.pad(edge_attr, ((0, EPAD - E), (0, 0)))
    pos4 = jnp.pad(pos, ((0, 0), (0, 1)))
    z32 = jnp.zeros((NPAD, TS), jnp.float32)
    Bmat = jnp.transpose(nW2.reshape(32, H, H), (2, 0, 1)).reshape(H, 32 * H)
    Rmat = jnp.repeat(jnp.eye(32, dtype=jnp.float32), H, axis=1)
    nb2T = nb2.reshape(H, H).T

    tableS, tableD = _h0(x, W1, b1, pos4)
    for l in range(L):
        hp_src, p_dst = _gather(tableS, tableD, srcI, dstI)
        scat = _edge_net(ea, hp_src, p_dst, nW1, nb1[None, :],
                         Bmat, Rmat, nb2T, Wc[l].reshape(1, H))
        agg2 = _scatter(scat, dstI, z32)
        tableS, tableD = _update(tableS, agg2, Wr[l], br[l])
    out = _set2set(tableS, batch.astype(jnp.int32), W_ih, W_hh, b_lstm,
                   Wo1, bo1, Wo2, bo2)
    return out.reshape(-1)


# speedup vs baseline: 3.3499x; 1.2301x over previous
import jax
import jax.numpy as jnp
from jax import lax
from jax.experimental import pallas as pl
from jax.experimental.pallas import tpu as pltpu
from jax.experimental.pallas import tpu_sc as plsc

N, E, DIN, H, L, BGRAPH, M, DEDGE = 10000, 160000, 128, 16, 4, 64, 3, 4

NW = 32
CHUNK = 128
NCH = 40
EPW = CHUNK * NCH
NG = 4
GE = EPW // NG
EPAD = NW * EPW
NPAD = 10016
RPW = NPAD // 16
EB = 2048
TS = 32
TD = 16


def _h0_body(x_ref, w_ref, b_ref, p_ref, ts_ref, td_ref):
    h = x_ref[...] @ w_ref[...] + b_ref[...]
    p4 = p_ref[...]
    z12 = jnp.zeros((N, 12), jnp.float32)
    ts_ref[...] = jnp.concatenate([h, p4, z12], axis=1)
    td_ref[...] = jnp.concatenate([p4, z12], axis=1)


def _h0(x, W1, b1, pos4):
    return pl.pallas_call(
        _h0_body,
        out_shape=[
            jax.ShapeDtypeStruct((N, TS), jnp.float32),
            jax.ShapeDtypeStruct((N, TD), jnp.float32),
        ],
    )(x, W1, b1[None, :], pos4)


def _gather_body(ts_hbm, td_hbm, srcI, dstI, hp_out, pd_out,
                 idx_s, idx_d, srows0, drows0, sem0, sem1):
    c = lax.axis_index("c")
    s = lax.axis_index("s")
    wid = s * 2 + c
    pltpu.sync_copy(srcI.at[wid], idx_s)
    pltpu.sync_copy(dstI.at[wid], idx_d)

    def issue(g, sbuf, dbuf, sem):
        pltpu.async_copy(ts_hbm.at[idx_s.at[g]], sbuf, sem)
        pltpu.async_copy(td_hbm.at[idx_d.at[g]], dbuf, sem)

    def drain_write(g, sbuf, dbuf, sem):
        base = wid * EPW + g * GE
        pltpu.make_async_copy(ts_hbm.at[idx_s.at[g]], sbuf, sem).wait()
        pltpu.make_async_copy(td_hbm.at[idx_d.at[g]], dbuf, sem).wait()
        pltpu.sync_copy(sbuf, hp_out.at[pl.ds(base, GE)])
        pltpu.sync_copy(dbuf, pd_out.at[pl.ds(base, GE)])

    def body(g, _):
        issue(g, srows0, drows0, sem0)
        drain_write(g, srows0, drows0, sem0)
        return 0

    lax.fori_loop(0, NG, body, 0)


def _gather(tableS, tableD, srcI, dstI):
    mesh = plsc.VectorSubcoreMesh(core_axis_name="c", subcore_axis_name="s")
    f = pl.kernel(
        _gather_body,
        mesh=mesh,
        compiler_params=pltpu.CompilerParams(use_tc_tiling_on_sc=False),
        out_type=[
            jax.ShapeDtypeStruct((EPAD, TS), jnp.float32),
            jax.ShapeDtypeStruct((EPAD, TD), jnp.float32),
        ],
        scratch_types=[
            pltpu.VMEM((NG, GE), jnp.int32),
            pltpu.VMEM((NG, GE), jnp.int32),
            pltpu.VMEM((GE, TS), jnp.float32),
            pltpu.VMEM((GE, TD), jnp.float32),
            pltpu.SemaphoreType.DMA,
            pltpu.SemaphoreType.DMA,
        ],
    )
    return f(tableS, tableD, srcI, dstI)


def _scatter_body(scat_hbm, dstI, z32, agg_out, acc, idx_d, sbuf0, sem0):
    c = lax.axis_index("c")
    s = lax.axis_index("s")
    wid = s * 2 + c
    pltpu.sync_copy(z32.at[pl.ds(s * RPW, RPW)], acc.at[pl.ds(s * RPW, RPW)])
    pltpu.sync_copy(dstI.at[wid], idx_d)
    plsc.subcore_barrier()

    def issue(g, buf, sem):
        base = wid * EPW + g * GE
        pltpu.async_copy(scat_hbm.at[pl.ds(base, GE)], buf, sem)

    def drain_add(g, buf, sem):
        base = wid * EPW + g * GE
        pltpu.make_async_copy(scat_hbm.at[pl.ds(base, GE)], buf, sem).wait()
        pltpu.sync_copy(buf, acc.at[idx_d.at[g]], add=True)

    def body(g, _):
        issue(g, sbuf0, sem0)
        drain_add(g, sbuf0, sem0)
        return 0

    lax.fori_loop(0, NG, body, 0)
    plsc.subcore_barrier()
    pltpu.sync_copy(acc.at[pl.ds(s * RPW, RPW)],
                    agg_out.at[c, pl.ds(s * RPW, RPW)])


def _scatter(scat, dstI, z32):
    mesh = plsc.VectorSubcoreMesh(core_axis_name="c", subcore_axis_name="s")
    f = pl.kernel(
        _scatter_body,
        mesh=mesh,
        compiler_params=pltpu.CompilerParams(use_tc_tiling_on_sc=False),
        out_type=[
            jax.ShapeDtypeStruct((2, NPAD, TS), jnp.float32),
        ],
        scratch_types=[
            pltpu.VMEM_SHARED((NPAD, TS), jnp.float32),
            pltpu.VMEM((NG, GE), jnp.int32),
            pltpu.VMEM((GE, TS), jnp.float32),
            pltpu.SemaphoreType.DMA,
        ],
    )
    return f(scat, dstI, z32)[0]


def _edge_body(ea_ref, hp_ref, pd_ref, nw1_ref, nb1_ref, bmat_ref,
               rmat_ref, nb2t_ref, wc_ref, scat_ref):
    gid = pl.program_id(0)
    eidx = gid * EB + lax.broadcasted_iota(jnp.int32, (EB, 1), 0)
    valid = (eidx < E).astype(jnp.float32)

    hp = hp_ref[...]
    hs = hp[:, 0:16]
    rel = hp[:, 16:20] - pd_ref[:, 0:4]
    dist = jnp.sqrt(jnp.sum(rel * rel, axis=1, keepdims=True) + 1e-12)
    pre = ea_ref[...] @ nw1_ref[0:4, :] + dist * nw1_ref[4:5, :] + nb1_ref[...]
    t = pre * (1.0 / (1.0 + jnp.exp(-pre)))
    me = (t @ rmat_ref[...]) * (hs @ bmat_ref[...])
    m = me[:, :256] + me[:, 256:]
    m = m[:, :128] + m[:, 128:]
    m = m[:, :64] + m[:, 64:]
    m = m[:, :32] + m[:, 32:]
    m = m[:, :16] + m[:, 16:]
    msg = (m + hs @ nb2t_ref[...]) * valid
    cw = jnp.sum(msg * wc_ref[...], axis=1, keepdims=True)
    col = lax.broadcasted_iota(jnp.int32, (1, 4), 1)
    rc = rel * cw + jnp.where(col == 3, 1.0, 0.0) * valid
    z12 = jnp.zeros((EB, 12), jnp.float32)
    scat_ref[...] = jnp.concatenate([msg, rc, z12], axis=1)


def _edge_net(ea, hp_src, p_dst, nW1, nb1, Bmat, Rmat, nb2T, wc):
    grid = EPAD // EB
    return pl.pallas_call(
        _edge_body,
        grid=(grid,),
        in_specs=[
            pl.BlockSpec((EB, 4), lambda i: (i, 0)),
            pl.BlockSpec((EB, TS), lambda i: (i, 0)),
            pl.BlockSpec((EB, TD), lambda i: (i, 0)),
            pl.BlockSpec((5, 32), lambda i: (0, 0)),
            pl.BlockSpec((1, 32), lambda i: (0, 0)),
            pl.BlockSpec((H, 512), lambda i: (0, 0)),
            pl.BlockSpec((32, 512), lambda i: (0, 0)),
            pl.BlockSpec((H, H), lambda i: (0, 0)),
            pl.BlockSpec((1, H), lambda i: (0, 0)),
        ],
        out_specs=[
            pl.BlockSpec((EB, TS), lambda i: (i, 0)),
        ],
        out_shape=[
            jax.ShapeDtypeStruct((EPAD, TS), jnp.float32),
        ],
    )(ea, hp_src, p_dst, nW1, nb1, Bmat, Rmat, nb2T, wc)[0]


def _update_body(ts_ref, agg_ref, wr_ref, br_ref, tso_ref, tdo_ref):
    acc = agg_ref[0, :N, :] + agg_ref[1, :N, :]
    agg = acc[:, 0:16]
    ps = acc[:, 16:20]
    deg = jnp.maximum(ps[:, 3:4], 1.0)
    h = ts_ref[:, 0:16]
    pos4 = ts_ref[:, 16:20]
    h_new = h + h @ wr_ref[...] + agg / deg + br_ref[...]
    col = lax.broadcasted_iota(jnp.int32, (1, 4), 1)
    mask = jnp.where(col < 3, 1.0, 0.0)
    p_new = pos4 + (ps * mask) / deg
    z12 = jnp.zeros((N, 12), jnp.float32)
    tso_ref[...] = jnp.concatenate([h_new, p_new, z12], axis=1)
    tdo_ref[...] = jnp.concatenate([p_new, z12], axis=1)


def _update(tableS, agg2, Wr_l, br_l):
    return pl.pallas_call(
        _update_body,
        out_shape=[
            jax.ShapeDtypeStruct((N, TS), jnp.float32),
            jax.ShapeDtypeStruct((N, TD), jnp.float32),
        ],
    )(tableS, agg2, Wr_l, br_l[None, :])


def _s2s_body(ts_ref, b_ref, wih_ref, whh_ref, bl_ref, wo1_ref, bo1_ref,
              wo2_ref, bo2_ref, out_ref):
    h = ts_ref[:, 0:16]
    bidx = b_ref[...]
    gcol = lax.broadcasted_iota(jnp.int32, (N, BGRAPH), 1)
    onehot = (bidx == gcol).astype(jnp.float32)

    def sig(v):
        return 1.0 / (1.0 + jnp.exp(-v))

    q_star = jnp.zeros((BGRAPH, 2 * H), jnp.float32)
    hs = jnp.zeros((BGRAPH, H), jnp.float32)
    cs = jnp.zeros((BGRAPH, H), jnp.float32)
    dn0 = (((0,), (0,)), ((), ()))
    for _ in range(M):
        gates = q_star @ wih_ref[...] + hs @ whh_ref[...] + bl_ref[...]
        i = gates[:, 0 * H:1 * H]
        f = gates[:, 1 * H:2 * H]
        g = gates[:, 2 * H:3 * H]
        o = gates[:, 3 * H:4 * H]
        cs = sig(f) * cs + sig(i) * jnp.tanh(g)
        hs = sig(o) * jnp.tanh(cs)
        qb = onehot @ hs
        escore = jnp.sum(h * qb, axis=1, keepdims=True)
        masked = jnp.where(onehot > 0.0, escore, -3.4e38)
        emax = jnp.max(masked, axis=0, keepdims=True)
        emax = jnp.where(emax < -1e37, 0.0, emax)
        a = jnp.exp(escore - onehot @ emax.T)
        asum = lax.dot_general(onehot, a, dn0)
        asum = jnp.where(asum > 0.0, asum, 1.0)
        anorm = a / (onehot @ asum)
        r = lax.dot_general(onehot, anorm * h, dn0)
        q_star = jnp.concatenate([hs, r], axis=1)
    u = q_star @ wo1_ref[...] + bo1_ref[...]
    u = u * sig(u)
    out_ref[...] = u @ wo2_ref[...] + bo2_ref[...]


def _set2set(tableS, batch, W_ih, W_hh, b_lstm, Wo1, bo1, Wo2, bo2):
    return pl.pallas_call(
        _s2s_body,
        out_shape=jax.ShapeDtypeStruct((BGRAPH, 1), jnp.float32),
    )(tableS, batch[:, None], W_ih, W_hh, b_lstm[None, :], Wo1, bo1[None, :],
      Wo2, bo2[None, :])


def kernel(x, edge_index, edge_attr, pos, batch, W1, b1, nW1, nb1, nW2, nb2,
           Wr, br, Wc, W_ih, W_hh, b_lstm, Wo1, bo1, Wo2, bo2):
    src = edge_index[0].astype(jnp.int32)
    dst = edge_index[1].astype(jnp.int32)
    srcI = jnp.pad(src, (0, EPAD - E)).reshape(NW, NG, GE)
    dstI = jnp.pad(dst, (0, EPAD - E)).reshape(NW, NG, GE)
    ea = jnp.pad(edge_attr, ((0, EPAD - E), (0, 0)))
    pos4 = jnp.pad(pos, ((0, 0), (0, 1)))
    z32 = jnp.zeros((NPAD, TS), jnp.float32)
    Bmat = jnp.transpose(nW2.reshape(32, H, H), (2, 0, 1)).reshape(H, 32 * H)
    Rmat = jnp.repeat(jnp.eye(32, dtype=jnp.float32), H, axis=1)
    nb2T = nb2.reshape(H, H).T

    tableS, tableD = _h0(x, W1, b1, pos4)
    for l in range(L):
        hp_src, p_dst = _gather(tableS, tableD, srcI, dstI)
        scat = _edge_net(ea, hp_src, p_dst, nW1, nb1[None, :],
                         Bmat, Rmat, nb2T, Wc[l].reshape(1, H))
        agg2 = _scatter(scat, dstI, z32)
        tableS, tableD = _update(tableS, agg2, Wr[l], br[l])
    out = _set2set(tableS, batch.astype(jnp.int32), W_ih, W_hh, b_lstm,
                   Wo1, bo1, Wo2, bo2)
    return out.reshape(-1)


# speedup vs baseline: 3.4138x; 1.0191x over previous
import jax
import jax.numpy as jnp
from jax import lax
from jax.experimental import pallas as pl
from jax.experimental.pallas import tpu as pltpu
from jax.experimental.pallas import tpu_sc as plsc

N, E, DIN, H, L, BGRAPH, M, DEDGE = 10000, 160000, 128, 16, 4, 64, 3, 4

NW = 32
CHUNK = 128
NCH = 40
EPW = CHUNK * NCH
NG = 4
GE = EPW // NG
EPAD = NW * EPW
NPAD = 10016
RPW = NPAD // 16
EB = 4096
TS = 32
TD = 16


def _h0_body(x_ref, w_ref, b_ref, p_ref, ts_ref, td_ref):
    h = x_ref[...] @ w_ref[...] + b_ref[...]
    p4 = p_ref[...]
    z12 = jnp.zeros((N, 12), jnp.float32)
    ts_ref[...] = jnp.concatenate([h, p4, z12], axis=1)
    td_ref[...] = jnp.concatenate([p4, z12], axis=1)


def _h0(x, W1, b1, pos4):
    return pl.pallas_call(
        _h0_body,
        out_shape=[
            jax.ShapeDtypeStruct((N, TS), jnp.float32),
            jax.ShapeDtypeStruct((N, TD), jnp.float32),
        ],
    )(x, W1, b1[None, :], pos4)


def _gather_body(ts_hbm, td_hbm, srcI, dstI, hp_out, pd_out,
                 idx_s, idx_d, srows0, drows0, sem0, sem1):
    c = lax.axis_index("c")
    s = lax.axis_index("s")
    wid = s * 2 + c
    pltpu.sync_copy(srcI.at[wid], idx_s)
    pltpu.sync_copy(dstI.at[wid], idx_d)

    def issue(g, sbuf, dbuf, sem):
        pltpu.async_copy(ts_hbm.at[idx_s.at[g]], sbuf, sem)
        pltpu.async_copy(td_hbm.at[idx_d.at[g]], dbuf, sem)

    def drain_write(g, sbuf, dbuf, sem):
        base = wid * EPW + g * GE
        pltpu.make_async_copy(ts_hbm.at[idx_s.at[g]], sbuf, sem).wait()
        pltpu.make_async_copy(td_hbm.at[idx_d.at[g]], dbuf, sem).wait()
        pltpu.sync_copy(sbuf, hp_out.at[pl.ds(base, GE)])
        pltpu.sync_copy(dbuf, pd_out.at[pl.ds(base, GE)])

    def body(g, _):
        issue(g, srows0, drows0, sem0)
        drain_write(g, srows0, drows0, sem0)
        return 0

    lax.fori_loop(0, NG, body, 0)


def _gather(tableS, tableD, srcI, dstI):
    mesh = plsc.VectorSubcoreMesh(core_axis_name="c", subcore_axis_name="s")
    f = pl.kernel(
        _gather_body,
        mesh=mesh,
        compiler_params=pltpu.CompilerParams(use_tc_tiling_on_sc=False),
        out_type=[
            jax.ShapeDtypeStruct((EPAD, TS), jnp.float32),
            jax.ShapeDtypeStruct((EPAD, TD), jnp.float32),
        ],
        scratch_types=[
            pltpu.VMEM((NG, GE), jnp.int32),
            pltpu.VMEM((NG, GE), jnp.int32),
            pltpu.VMEM((GE, TS), jnp.float32),
            pltpu.VMEM((GE, TD), jnp.float32),
            pltpu.SemaphoreType.DMA,
            pltpu.SemaphoreType.DMA,
        ],
    )
    return f(tableS, tableD, srcI, dstI)


def _scatter_body(scat_hbm, dstI, z32, agg_out, acc, idx_d, sbuf0, sem0):
    c = lax.axis_index("c")
    s = lax.axis_index("s")
    wid = s * 2 + c
    pltpu.sync_copy(z32.at[pl.ds(s * RPW, RPW)], acc.at[pl.ds(s * RPW, RPW)])
    pltpu.sync_copy(dstI.at[wid], idx_d)
    plsc.subcore_barrier()

    def issue(g, buf, sem):
        base = wid * EPW + g * GE
        pltpu.async_copy(scat_hbm.at[pl.ds(base, GE)], buf, sem)

    def drain_add(g, buf, sem):
        base = wid * EPW + g * GE
        pltpu.make_async_copy(scat_hbm.at[pl.ds(base, GE)], buf, sem).wait()
        pltpu.sync_copy(buf, acc.at[idx_d.at[g]], add=True)

    def body(g, _):
        issue(g, sbuf0, sem0)
        drain_add(g, sbuf0, sem0)
        return 0

    lax.fori_loop(0, NG, body, 0)
    plsc.subcore_barrier()
    pltpu.sync_copy(acc.at[pl.ds(s * RPW, RPW)],
                    agg_out.at[c, pl.ds(s * RPW, RPW)])


def _scatter(scat, dstI, z32):
    mesh = plsc.VectorSubcoreMesh(core_axis_name="c", subcore_axis_name="s")
    f = pl.kernel(
        _scatter_body,
        mesh=mesh,
        compiler_params=pltpu.CompilerParams(use_tc_tiling_on_sc=False),
        out_type=[
            jax.ShapeDtypeStruct((2, NPAD, TS), jnp.float32),
        ],
        scratch_types=[
            pltpu.VMEM_SHARED((NPAD, TS), jnp.float32),
            pltpu.VMEM((NG, GE), jnp.int32),
            pltpu.VMEM((GE, TS), jnp.float32),
            pltpu.SemaphoreType.DMA,
        ],
    )
    return f(scat, dstI, z32)[0]


def _edge_body(ea_ref, hp_ref, pd_ref, nw1_ref, nb1_ref, bmat_ref,
               rmat_ref, nb2t_ref, wc_ref, scat_ref):
    gid = pl.program_id(0)
    eidx = gid * EB + lax.broadcasted_iota(jnp.int32, (EB, 1), 0)
    valid = (eidx < E).astype(jnp.float32)

    hp = hp_ref[...]
    hs = hp[:, 0:16]
    rel = hp[:, 16:20] - pd_ref[:, 0:4]
    dist = jnp.sqrt(jnp.sum(rel * rel, axis=1, keepdims=True) + 1e-12)
    pre = ea_ref[...] @ nw1_ref[0:4, :] + dist * nw1_ref[4:5, :] + nb1_ref[...]
    t = pre * (1.0 / (1.0 + jnp.exp(-pre)))
    me = (t @ rmat_ref[...]) * (hs @ bmat_ref[...])
    m = me[:, :256] + me[:, 256:]
    m = m[:, :128] + m[:, 128:]
    m = m[:, :64] + m[:, 64:]
    m = m[:, :32] + m[:, 32:]
    m = m[:, :16] + m[:, 16:]
    msg = (m + hs @ nb2t_ref[...]) * valid
    cw = jnp.sum(msg * wc_ref[...], axis=1, keepdims=True)
    col = lax.broadcasted_iota(jnp.int32, (1, 4), 1)
    rc = rel * cw + jnp.where(col == 3, 1.0, 0.0) * valid
    z12 = jnp.zeros((EB, 12), jnp.float32)
    scat_ref[...] = jnp.concatenate([msg, rc, z12], axis=1)


def _edge_net(ea, hp_src, p_dst, nW1, nb1, Bmat, Rmat, nb2T, wc):
    grid = EPAD // EB
    return pl.pallas_call(
        _edge_body,
        grid=(grid,),
        in_specs=[
            pl.BlockSpec((EB, 4), lambda i: (i, 0)),
            pl.BlockSpec((EB, TS), lambda i: (i, 0)),
            pl.BlockSpec((EB, TD), lambda i: (i, 0)),
            pl.BlockSpec((5, 32), lambda i: (0, 0)),
            pl.BlockSpec((1, 32), lambda i: (0, 0)),
            pl.BlockSpec((H, 512), lambda i: (0, 0)),
            pl.BlockSpec((32, 512), lambda i: (0, 0)),
            pl.BlockSpec((H, H), lambda i: (0, 0)),
            pl.BlockSpec((1, H), lambda i: (0, 0)),
        ],
        out_specs=[
            pl.BlockSpec((EB, TS), lambda i: (i, 0)),
        ],
        out_shape=[
            jax.ShapeDtypeStruct((EPAD, TS), jnp.float32),
        ],
    )(ea, hp_src, p_dst, nW1, nb1, Bmat, Rmat, nb2T, wc)[0]


def _update_body(ts_ref, agg_ref, wr_ref, br_ref, tso_ref, tdo_ref):
    acc = agg_ref[0, :N, :] + agg_ref[1, :N, :]
    agg = acc[:, 0:16]
    ps = acc[:, 16:20]
    deg = jnp.maximum(ps[:, 3:4], 1.0)
    h = ts_ref[:, 0:16]
    pos4 = ts_ref[:, 16:20]
    h_new = h + h @ wr_ref[...] + agg / deg + br_ref[...]
    col = lax.broadcasted_iota(jnp.int32, (1, 4), 1)
    mask = jnp.where(col < 3, 1.0, 0.0)
    p_new = pos4 + (ps * mask) / deg
    z12 = jnp.zeros((N, 12), jnp.float32)
    tso_ref[...] = jnp.concatenate([h_new, p_new, z12], axis=1)
    tdo_ref[...] = jnp.concatenate([p_new, z12], axis=1)


def _update(tableS, agg2, Wr_l, br_l):
    return pl.pallas_call(
        _update_body,
        out_shape=[
            jax.ShapeDtypeStruct((N, TS), jnp.float32),
            jax.ShapeDtypeStruct((N, TD), jnp.float32),
        ],
    )(tableS, agg2, Wr_l, br_l[None, :])


def _s2s_body(ts_ref, b_ref, wih_ref, whh_ref, bl_ref, wo1_ref, bo1_ref,
              wo2_ref, bo2_ref, out_ref):
    h = ts_ref[:, 0:16]
    bidx = b_ref[...]
    gcol = lax.broadcasted_iota(jnp.int32, (N, BGRAPH), 1)
    onehot = (bidx == gcol).astype(jnp.float32)

    def sig(v):
        return 1.0 / (1.0 + jnp.exp(-v))

    q_star = jnp.zeros((BGRAPH, 2 * H), jnp.float32)
    hs = jnp.zeros((BGRAPH, H), jnp.float32)
    cs = jnp.zeros((BGRAPH, H), jnp.float32)
    dn0 = (((0,), (0,)), ((), ()))
    for _ in range(M):
        gates = q_star @ wih_ref[...] + hs @ whh_ref[...] + bl_ref[...]
        i = gates[:, 0 * H:1 * H]
        f = gates[:, 1 * H:2 * H]
        g = gates[:, 2 * H:3 * H]
        o = gates[:, 3 * H:4 * H]
        cs = sig(f) * cs + sig(i) * jnp.tanh(g)
        hs = sig(o) * jnp.tanh(cs)
        qb = onehot @ hs
        escore = jnp.sum(h * qb, axis=1, keepdims=True)
        masked = jnp.where(onehot > 0.0, escore, -3.4e38)
        emax = jnp.max(masked, axis=0, keepdims=True)
        emax = jnp.where(emax < -1e37, 0.0, emax)
        a = jnp.exp(escore - onehot @ emax.T)
        asum = lax.dot_general(onehot, a, dn0)
        asum = jnp.where(asum > 0.0, asum, 1.0)
        anorm = a / (onehot @ asum)
        r = lax.dot_general(onehot, anorm * h, dn0)
        q_star = jnp.concatenate([hs, r], axis=1)
    u = q_star @ wo1_ref[...] + bo1_ref[...]
    u = u * sig(u)
    out_ref[...] = u @ wo2_ref[...] + bo2_ref[...]


def _set2set(tableS, batch, W_ih, W_hh, b_lstm, Wo1, bo1, Wo2, bo2):
    return pl.pallas_call(
        _s2s_body,
        out_shape=jax.ShapeDtypeStruct((BGRAPH, 1), jnp.float32),
    )(tableS, batch[:, None], W_ih, W_hh, b_lstm[None, :], Wo1, bo1[None, :],
      Wo2, bo2[None, :])


def kernel(x, edge_index, edge_attr, pos, batch, W1, b1, nW1, nb1, nW2, nb2,
           Wr, br, Wc, W_ih, W_hh, b_lstm, Wo1, bo1, Wo2, bo2):
    src = edge_index[0].astype(jnp.int32)
    dst = edge_index[1].astype(jnp.int32)
    srcI = jnp.pad(src, (0, EPAD - E)).reshape(NW, NG, GE)
    dstI = jnp.pad(dst, (0, EPAD - E)).reshape(NW, NG, GE)
    ea = jnp.pad(edge_attr, ((0, EPAD - E), (0, 0)))
    pos4 = jnp.pad(pos, ((0, 0), (0, 1)))
    z32 = jnp.zeros((NPAD, TS), jnp.float32)
    Bmat = jnp.transpose(nW2.reshape(32, H, H), (2, 0, 1)).reshape(H, 32 * H)
    Rmat = jnp.repeat(jnp.eye(32, dtype=jnp.float32), H, axis=1)
    nb2T = nb2.reshape(H, H).T

    tableS, tableD = _h0(x, W1, b1, pos4)
    for l in range(L):
        hp_src, p_dst = _gather(tableS, tableD, srcI, dstI)
        scat = _edge_net(ea, hp_src, p_dst, nW1, nb1[None, :],
                         Bmat, Rmat, nb2T, Wc[l].reshape(1, H))
        agg2 = _scatter(scat, dstI, z32)
        tableS, tableD = _update(tableS, agg2, Wr[l], br[l])
    out = _set2set(tableS, batch.astype(jnp.int32), W_ih, W_hh, b_lstm,
                   Wo1, bo1, Wo2, bo2)
    return out.reshape(-1)
